# baseline (speedup 1.0000x reference)
import functools

import jax
import jax.numpy as jnp
from jax import lax
from jax.experimental import pallas as pl
from jax.experimental.pallas import tpu as pltpu
from jax.experimental.pallas import tpu_sc as plsc

HIDDEN = 128
NC = 2
NS = 16
CHUNK = 64
NBUF = 8
LAG = 3
SPLIT = (88, 112)


def _make_gather(n_rows: int):
    total_chunks = n_rows // CHUNK
    assert n_rows % CHUNK == 0
    assert SPLIT[0] + SPLIT[1] == total_chunks // NS
    assert SPLIT[0] % NBUF == 0 and SPLIT[1] % NBUF == 0
    mesh = plsc.VectorSubcoreMesh(core_axis_name="c", subcore_axis_name="s")

    scratch = [pltpu.VMEM((max(SPLIT), CHUNK), jnp.int32)]
    scratch += [pltpu.VMEM((CHUNK, HIDDEN), jnp.float32) for _ in range(NBUF)]
    scratch += [pltpu.SemaphoreType.DMA for _ in range(2 * NBUF)]

    @functools.partial(
        pl.kernel,
        mesh=mesh,
        out_type=jax.ShapeDtypeStruct((n_rows, HIDDEN), jnp.float32),
        scratch_types=scratch,
    )
    def gather_kernel(idx_hbm, table_hbm, out_hbm, idx_v, *bufs_and_sems):
        rows = bufs_and_sems[:NBUF]
        gsem = bufs_and_sems[NBUF:2 * NBUF]
        wsem = bufs_and_sems[2 * NBUF:]
        cid = lax.axis_index("c")
        sid = lax.axis_index("s")

        def pipeline(chunk_base, nchunks):
            base = chunk_base * CHUNK
            nrounds = nchunks // NBUF
            pltpu.sync_copy(idx_hbm.at[pl.ds(chunk_base, nchunks)],
                            idx_v.at[pl.ds(0, nchunks)])

            def gather_cp(g, b):
                return pltpu.make_async_copy(
                    table_hbm.at[idx_v.at[g]], rows[b], gsem[b])

            def write_cp(g, b):
                dst = out_hbm.at[pl.ds(base + g * CHUNK, CHUNK)]
                return pltpu.make_async_copy(rows[b], dst, wsem[b])

            for b in range(NBUF):
                gather_cp(b, b).start()

            def round_body(r, carry):
                g0 = r * NBUF
                for b in range(NBUF):
                    gather_cp(g0 + b, b).wait()
                    write_cp(g0 + b, b).start()
                    if b >= LAG:
                        bb = b - LAG
                        write_cp(g0 + bb, bb).wait()
                        gather_cp(g0 + NBUF + bb, bb).start()
                for bb in range(NBUF - LAG, NBUF):
                    write_cp(g0 + bb, bb).wait()
                    gather_cp(g0 + NBUF + bb, bb).start()
                return carry

            lax.fori_loop(0, nrounds - 1, round_body, 0)

            g0 = (nrounds - 1) * NBUF
            for b in range(NBUF):
                gather_cp(g0 + b, b).wait()
                write_cp(g0 + b, b).start()
            for b in range(NBUF):
                write_cp(g0 + b, b).wait()

        @pl.when(cid == 0)
        def _():
            pipeline(sid * SPLIT[0], SPLIT[0])

        @pl.when(cid == 1)
        def _():
            pipeline(NS * SPLIT[0] + sid * SPLIT[1], SPLIT[1])

    return gather_kernel


def kernel(input_ids, table):
    batch, seq = input_ids.shape
    n_rows = batch * seq
    idx = input_ids.reshape(n_rows // CHUNK, CHUNK).astype(jnp.int32)
    out = _make_gather(n_rows)(idx, table)
    return out.reshape(batch, seq, HIDDEN)


# speedup vs baseline: 1.0194x; 1.0194x over previous
import functools

import jax
import jax.numpy as jnp
from jax import lax
from jax.experimental import pallas as pl
from jax.experimental.pallas import tpu as pltpu
from jax.experimental.pallas import tpu_sc as plsc

HIDDEN = 128
NC = 2
NS = 16
NW = NC * NS
CHUNK = 64
NBUF = 10


def _make_gather(n_rows: int):
    assert n_rows % (NW * CHUNK) == 0
    b_per_w = n_rows // NW
    nchunks = b_per_w // CHUNK
    assert nchunks % NBUF == 0
    nrounds = nchunks // NBUF
    mesh = plsc.VectorSubcoreMesh(core_axis_name="c", subcore_axis_name="s")

    scratch = [pltpu.VMEM((nchunks, CHUNK), jnp.int32)]
    scratch += [pltpu.VMEM((CHUNK, HIDDEN), jnp.float32) for _ in range(NBUF)]
    scratch += [pltpu.SemaphoreType.DMA for _ in range(2 * NBUF)]

    @functools.partial(
        pl.kernel,
        mesh=mesh,
        out_type=jax.ShapeDtypeStruct((n_rows, HIDDEN), jnp.float32),
        scratch_types=scratch,
    )
    def gather_kernel(idx_hbm, table_hbm, out_hbm, idx_v, *bufs_and_sems):
        rows = bufs_and_sems[:NBUF]
        gsem = bufs_and_sems[NBUF:2 * NBUF]
        wsem = bufs_and_sems[2 * NBUF:]
        wid = lax.axis_index("s") * NC + lax.axis_index("c")
        base = wid * b_per_w
        pltpu.sync_copy(idx_hbm.at[wid], idx_v)

        def gather_cp(g, b):
            return pltpu.make_async_copy(table_hbm.at[idx_v.at[g]], rows[b], gsem[b])

        def write_cp(g, b):
            dst = out_hbm.at[pl.ds(base + g * CHUNK, CHUNK)]
            return pltpu.make_async_copy(rows[b], dst, wsem[b])

        for b in range(NBUF):
            gather_cp(b, b).start()

        LAG = 2

        def round_body(r, carry):
            g0 = r * NBUF
            for b in range(NBUF):
                gather_cp(g0 + b, b).wait()
                write_cp(g0 + b, b).start()
                if b >= LAG:
                    bb = b - LAG
                    write_cp(g0 + bb, bb).wait()
                    gather_cp(g0 + NBUF + bb, bb).start()
            for bb in range(NBUF - LAG, NBUF):
                write_cp(g0 + bb, bb).wait()
                gather_cp(g0 + NBUF + bb, bb).start()
            return carry

        lax.fori_loop(0, nrounds - 1, round_body, 0)

        g0 = (nrounds - 1) * NBUF
        for b in range(NBUF):
            gather_cp(g0 + b, b).wait()
            write_cp(g0 + b, b).start()
        for b in range(NBUF):
            write_cp(g0 + b, b).wait()

    return gather_kernel


def kernel(input_ids, table):
    batch, seq = input_ids.shape
    n_rows = batch * seq
    idx = input_ids.reshape(NW, n_rows // (NW * CHUNK), CHUNK).astype(jnp.int32)
    out = _make_gather(n_rows)(idx, table)
    return out.reshape(batch, seq, HIDDEN)


# speedup vs baseline: 1.0266x; 1.0071x over previous
import functools

import jax
import jax.numpy as jnp
from jax import lax
from jax.experimental import pallas as pl
from jax.experimental.pallas import tpu as pltpu
from jax.experimental.pallas import tpu_sc as plsc

HIDDEN = 128
NC = 2
NS = 16
NW = NC * NS
CHUNK = 80
NBUF = 10


def _make_gather(n_rows: int):
    assert n_rows % (NW * CHUNK) == 0
    b_per_w = n_rows // NW
    nchunks = b_per_w // CHUNK
    assert nchunks % NBUF == 0
    nrounds = nchunks // NBUF
    mesh = plsc.VectorSubcoreMesh(core_axis_name="c", subcore_axis_name="s")

    scratch = [pltpu.VMEM((nchunks, CHUNK), jnp.int32)]
    scratch += [pltpu.VMEM((CHUNK, HIDDEN), jnp.float32) for _ in range(NBUF)]
    scratch += [pltpu.SemaphoreType.DMA for _ in range(2 * NBUF)]

    @functools.partial(
        pl.kernel,
        mesh=mesh,
        out_type=jax.ShapeDtypeStruct((n_rows, HIDDEN), jnp.float32),
        scratch_types=scratch,
    )
    def gather_kernel(idx_hbm, table_hbm, out_hbm, idx_v, *bufs_and_sems):
        rows = bufs_and_sems[:NBUF]
        gsem = bufs_and_sems[NBUF:2 * NBUF]
        wsem = bufs_and_sems[2 * NBUF:]
        wid = lax.axis_index("s") * NC + lax.axis_index("c")
        base = wid * b_per_w
        pltpu.sync_copy(idx_hbm.at[wid], idx_v)

        def gather_cp(g, b):
            return pltpu.make_async_copy(table_hbm.at[idx_v.at[g]], rows[b], gsem[b])

        def write_cp(g, b):
            dst = out_hbm.at[pl.ds(base + g * CHUNK, CHUNK)]
            return pltpu.make_async_copy(rows[b], dst, wsem[b])

        for b in range(NBUF):
            gather_cp(b, b).start()

        LAG = 2

        def round_body(r, carry):
            g0 = r * NBUF
            for b in range(NBUF):
                gather_cp(g0 + b, b).wait()
                write_cp(g0 + b, b).start()
                if b >= LAG:
                    bb = b - LAG
                    write_cp(g0 + bb, bb).wait()
                    gather_cp(g0 + NBUF + bb, bb).start()
            for bb in range(NBUF - LAG, NBUF):
                write_cp(g0 + bb, bb).wait()
                gather_cp(g0 + NBUF + bb, bb).start()
            return carry

        lax.fori_loop(0, nrounds - 1, round_body, 0)

        g0 = (nrounds - 1) * NBUF
        for b in range(NBUF):
            gather_cp(g0 + b, b).wait()
            write_cp(g0 + b, b).start()
        for b in range(NBUF):
            write_cp(g0 + b, b).wait()

    return gather_kernel


def kernel(input_ids, table):
    batch, seq = input_ids.shape
    n_rows = batch * seq
    idx = input_ids.reshape(NW, n_rows // (NW * CHUNK), CHUNK).astype(jnp.int32)
    out = _make_gather(n_rows)(idx, table)
    return out.reshape(batch, seq, HIDDEN)
